# baseline (speedup 1.0000x reference)
import functools

import jax
import jax.numpy as jnp
from jax import lax
from jax.experimental import pallas as pl
from jax.experimental.pallas import tpu as pltpu
from jax.experimental.pallas import tpu_sc as plsc

K = 8
B = 100000
PROJ_DIM = 32
EMB_DIM = 32
HIDDEN = 128
BATCH = 16384

NC = 2
NS = 16
NW = NC * NS
PER_W = BATCH // NW
C = 128
NCHUNK = PER_W // C


def _sc_gather_sum(tab_hbm, idx_hbm, z_hbm, idx_v, rows_v, z_v, sem):
    c = lax.axis_index("c")
    s = lax.axis_index("s")
    wid = s * NC + c

    def chunk_body(ci, carry):
        blk = (wid * NCHUNK + ci) * K
        pltpu.sync_copy(idx_hbm.at[pl.ds(blk, K)], idx_v)

        copies = [
            pltpu.async_copy(tab_hbm.at[k].at[idx_v.at[k]], rows_v.at[k], sem)
            for k in range(K)
        ]
        for cop in copies:
            cop.wait()

        def sum_body(r, _):
            for v in range(PROJ_DIM // 16):
                sl = pl.ds(v * 16, 16)
                acc = rows_v[0, r, sl]
                for k in range(1, K):
                    acc = acc + rows_v[k, r, sl]
                z_v[r, sl] = acc
            return 0

        lax.fori_loop(0, C, sum_body, 0)

        row0 = wid * PER_W + ci * C
        pltpu.sync_copy(z_v, z_hbm.at[pl.ds(row0, C)])
        return carry

    lax.fori_loop(0, NCHUNK, chunk_body, 0)


@jax.jit
def _gather_sum(tables, idx):
    mesh = plsc.VectorSubcoreMesh(
        core_axis_name="c", subcore_axis_name="s", num_cores=NC, num_subcores=NS
    )
    return pl.kernel(
        _sc_gather_sum,
        out_type=jax.ShapeDtypeStruct((BATCH, PROJ_DIM), jnp.float32),
        mesh=mesh,
        scratch_types=[
            pltpu.VMEM((K, C), jnp.int32),
            pltpu.VMEM((K, C, PROJ_DIM), jnp.float32),
            pltpu.VMEM((C, PROJ_DIM), jnp.float32),
            pltpu.SemaphoreType.DMA,
        ],
        compiler_params=pltpu.CompilerParams(use_tc_tiling_on_sc=False),
    )(tables, idx)


TB = 2048


def _mlp_body(z_ref, w1_ref, b1_ref, w2_ref, b2_ref, o_ref):
    h = jnp.dot(z_ref[...], w1_ref[...], preferred_element_type=jnp.float32)
    h = jnp.maximum(h + b1_ref[...], 0.0)
    o = jnp.dot(h, w2_ref[...], preferred_element_type=jnp.float32)
    o_ref[...] = o + b2_ref[...]


@jax.jit
def _mlp(z, W1, b1, W2, b2):
    return pl.pallas_call(
        _mlp_body,
        grid=(BATCH // TB,),
        in_specs=[
            pl.BlockSpec((TB, PROJ_DIM), lambda i: (i, 0)),
            pl.BlockSpec((PROJ_DIM, HIDDEN), lambda i: (0, 0)),
            pl.BlockSpec((1, HIDDEN), lambda i: (0, 0)),
            pl.BlockSpec((HIDDEN, EMB_DIM), lambda i: (0, 0)),
            pl.BlockSpec((1, EMB_DIM), lambda i: (0, 0)),
        ],
        out_specs=pl.BlockSpec((TB, EMB_DIM), lambda i: (i, 0)),
        out_shape=jax.ShapeDtypeStruct((BATCH, EMB_DIM), jnp.float32),
    )(z, W1, b1.reshape(1, HIDDEN), W2, b2.reshape(1, EMB_DIM))


def kernel(buckets, tables, W1, b1, W2, b2):
    idx = (
        buckets.reshape(NW, NCHUNK, C, K)
        .transpose(0, 1, 3, 2)
        .reshape(NW * NCHUNK * K, C)
        .astype(jnp.int32)
    )
    z = _gather_sum(tables, idx)
    return _mlp(z, W1, b1, W2, b2)


# speedup vs baseline: 1.0009x; 1.0009x over previous
import jax
import jax.numpy as jnp
from jax import lax
from jax.experimental import pallas as pl
from jax.experimental.pallas import tpu as pltpu
from jax.experimental.pallas import tpu_sc as plsc

K = 8
B = 100000
PROJ_DIM = 32
EMB_DIM = 32
HIDDEN = 128
BATCH = 16384

NC = 2
NS = 16
NW = NC * NS
PER_W = BATCH // NW
C = 128
NCHUNK = PER_W // C
L = 16


def _sc_gather_sum(tab_hbm, bkt_hbm, z_hbm, bidx_v, idx_v, rows_v, z_v,
                   sem_a, sem_b):
    c = lax.axis_index("c")
    s = lax.axis_index("s")
    wid = s * NC + c
    base = wid * PER_W
    sems = (sem_a, sem_b)
    lanes = lax.iota(jnp.int32, L)

    def stage(ci, buf):
        pltpu.sync_copy(bkt_hbm.at[pl.ds(base + ci * C, C)], bidx_v.at[buf])
        for k in range(K):
            kcol = jnp.full((L,), k, jnp.int32)
            for g in range(C // L):
                vals = plsc.load_gather(
                    bidx_v.at[buf], [lanes + (g * L), kcol]
                )
                idx_v[buf, k, pl.ds(g * L, L)] = vals
        return [
            pltpu.async_copy(
                tab_hbm.at[k].at[idx_v.at[buf].at[k]],
                rows_v.at[buf].at[k],
                sems[buf],
            )
            for k in range(K)
        ]

    def reduce_store(ci, buf, copies):
        for cop in copies:
            cop.wait()

        def sum_body(r2, _):
            for u in range(2):
                r = r2 * 2 + u
                for v in range(PROJ_DIM // L):
                    sl = pl.ds(v * L, L)
                    acc = rows_v[buf, 0, r, sl]
                    for k in range(1, K):
                        acc = acc + rows_v[buf, k, r, sl]
                    z_v[r, sl] = acc
            return 0

        lax.fori_loop(0, C // 2, sum_body, 0)
        pltpu.sync_copy(z_v, z_hbm.at[pl.ds(base + ci * C, C)])

    copies = stage(0, 0)
    for ci in range(NCHUNK):
        buf = ci % 2
        nxt = None
        if ci + 1 < NCHUNK:
            nxt = stage(ci + 1, (ci + 1) % 2)
        reduce_store(ci, buf, copies)
        copies = nxt


@jax.jit
def _gather_sum(tables, buckets):
    mesh = plsc.VectorSubcoreMesh(
        core_axis_name="c", subcore_axis_name="s", num_cores=NC, num_subcores=NS
    )
    return pl.kernel(
        _sc_gather_sum,
        out_type=jax.ShapeDtypeStruct((BATCH, PROJ_DIM), jnp.float32),
        mesh=mesh,
        scratch_types=[
            pltpu.VMEM((2, C, K), jnp.int32),
            pltpu.VMEM((2, K, C), jnp.int32),
            pltpu.VMEM((2, K, C, PROJ_DIM), jnp.float32),
            pltpu.VMEM((C, PROJ_DIM), jnp.float32),
            pltpu.SemaphoreType.DMA,
            pltpu.SemaphoreType.DMA,
        ],
        compiler_params=pltpu.CompilerParams(use_tc_tiling_on_sc=False, needs_layout_passes=False),
    )(tables, buckets)


TB = 2048


def _mlp_body(z_ref, w1_ref, b1_ref, w2_ref, b2_ref, o_ref):
    h = jnp.dot(z_ref[...], w1_ref[...], preferred_element_type=jnp.float32)
    h = jnp.maximum(h + b1_ref[...], 0.0)
    o = jnp.dot(h, w2_ref[...], preferred_element_type=jnp.float32)
    o_ref[...] = o + b2_ref[...]


@jax.jit
def _mlp(z, W1, b1, W2, b2):
    return pl.pallas_call(
        _mlp_body,
        grid=(BATCH // TB,),
        in_specs=[
            pl.BlockSpec((TB, PROJ_DIM), lambda i: (i, 0)),
            pl.BlockSpec((PROJ_DIM, HIDDEN), lambda i: (0, 0)),
            pl.BlockSpec((1, HIDDEN), lambda i: (0, 0)),
            pl.BlockSpec((HIDDEN, EMB_DIM), lambda i: (0, 0)),
            pl.BlockSpec((1, EMB_DIM), lambda i: (0, 0)),
        ],
        out_specs=pl.BlockSpec((TB, EMB_DIM), lambda i: (i, 0)),
        out_shape=jax.ShapeDtypeStruct((BATCH, EMB_DIM), jnp.float32),
    )(z, W1, b1.reshape(1, HIDDEN), W2, b2.reshape(1, EMB_DIM))


def kernel(buckets, tables, W1, b1, W2, b2):
    z = _gather_sum(tables, buckets)
    return _mlp(z, W1, b1, W2, b2)


# speedup vs baseline: 2.3521x; 2.3499x over previous
import jax
import jax.numpy as jnp
from jax import lax
from jax.experimental import pallas as pl
from jax.experimental.pallas import tpu as pltpu
from jax.experimental.pallas import tpu_sc as plsc

K = 8
B = 100000
PROJ_DIM = 32
EMB_DIM = 32
HIDDEN = 128
BATCH = 16384

NC = 2
NS = 16
NW = NC * NS
L = 16
HB = 4096


def _sc_gather_sum(tab_hbm, bkt_hbm, zt_hbm, v_ref, bidx_v, z_v, sem):
    c = lax.axis_index("c")
    s = lax.axis_index("s")
    j = s * NC + c

    for k in range(K):
        pltpu.sync_copy(tab_hbm.at[k, j], v_ref)
        for h in range(BATCH // HB):
            pltpu.sync_copy(bkt_hbm.at[k, pl.ds(h * HB, HB)], bidx_v)

            def gather_body(g, _, k=k, h=h):
                idx = bidx_v[pl.ds(g * L, L)]
                vals = plsc.load_gather(v_ref, [idx])
                zsl = pl.ds(h * HB + g * L, L)
                if k == 0:
                    z_v[zsl] = vals
                else:
                    z_v[zsl] = z_v[zsl] + vals
                return 0

            lax.fori_loop(0, HB // L, gather_body, 0)

    pltpu.sync_copy(z_v, zt_hbm.at[j])


@jax.jit
def _gather_sum(tables_t, buckets_t):
    mesh = plsc.VectorSubcoreMesh(
        core_axis_name="c", subcore_axis_name="s", num_cores=NC, num_subcores=NS
    )
    return pl.kernel(
        _sc_gather_sum,
        out_type=jax.ShapeDtypeStruct((PROJ_DIM, BATCH), jnp.float32),
        mesh=mesh,
        scratch_types=[
            pltpu.VMEM((B,), jnp.float32),
            pltpu.VMEM((HB,), jnp.int32),
            pltpu.VMEM((BATCH,), jnp.float32),
            pltpu.SemaphoreType.DMA,
        ],
        compiler_params=pltpu.CompilerParams(
            use_tc_tiling_on_sc=True, needs_layout_passes=False
        ),
    )(tables_t, buckets_t)


TB = 2048


def _mlp_body(zt_ref, w1_ref, b1_ref, w2_ref, b2_ref, o_ref):
    h = lax.dot_general(
        zt_ref[...], w1_ref[...], (((0,), (0,)), ((), ())),
        preferred_element_type=jnp.float32,
    )
    h = jnp.maximum(h + b1_ref[...], 0.0)
    o = jnp.dot(h, w2_ref[...], preferred_element_type=jnp.float32)
    o_ref[...] = o + b2_ref[...]


@jax.jit
def _mlp(zt, W1, b1, W2, b2):
    return pl.pallas_call(
        _mlp_body,
        grid=(BATCH // TB,),
        in_specs=[
            pl.BlockSpec((PROJ_DIM, TB), lambda i: (0, i)),
            pl.BlockSpec((PROJ_DIM, HIDDEN), lambda i: (0, 0)),
            pl.BlockSpec((1, HIDDEN), lambda i: (0, 0)),
            pl.BlockSpec((HIDDEN, EMB_DIM), lambda i: (0, 0)),
            pl.BlockSpec((1, EMB_DIM), lambda i: (0, 0)),
        ],
        out_specs=pl.BlockSpec((TB, EMB_DIM), lambda i: (i, 0)),
        out_shape=jax.ShapeDtypeStruct((BATCH, EMB_DIM), jnp.float32),
    )(zt, W1, b1.reshape(1, HIDDEN), W2, b2.reshape(1, EMB_DIM))


def kernel(buckets, tables, W1, b1, W2, b2):
    tables_t = tables.transpose(0, 2, 1)
    buckets_t = buckets.T
    zt = _gather_sum(tables_t, buckets_t)
    return _mlp(zt, W1, b1, W2, b2)


# speedup vs baseline: 2.6469x; 1.1253x over previous
import jax
import jax.numpy as jnp
from jax import lax
from jax.experimental import pallas as pl
from jax.experimental.pallas import tpu as pltpu
from jax.experimental.pallas import tpu_sc as plsc

K = 8
B = 100000
PROJ_DIM = 32
EMB_DIM = 32
HIDDEN = 128
BATCH = 16384

NC = 2
NS = 16
NW = NC * NS
L = 16
HB = 8192
UNROLL = 4


def _sc_gather_sum(tab_hbm, bkt_hbm, zt_hbm, v_ref, bidx_v, z_v, sem):
    c = lax.axis_index("c")
    s = lax.axis_index("s")
    j = s * NC + c

    for k in range(K):
        pltpu.sync_copy(tab_hbm.at[k, j], v_ref)
        for h in range(BATCH // HB):
            pltpu.sync_copy(bkt_hbm.at[k, pl.ds(h * HB, HB)], bidx_v)

            def gather_body(g0, _, k=k, h=h):
                for u in range(UNROLL):
                    g = g0 * UNROLL + u
                    idx = bidx_v[pl.ds(g * L, L)]
                    vals = plsc.load_gather(v_ref, [idx])
                    zsl = pl.ds(h * HB + g * L, L)
                    if k == 0:
                        z_v[zsl] = vals
                    else:
                        z_v[zsl] = z_v[zsl] + vals
                return 0

            lax.fori_loop(0, HB // L // UNROLL, gather_body, 0)

    pltpu.sync_copy(z_v, zt_hbm.at[j])


@jax.jit
def _gather_sum(tables_t, buckets_t):
    mesh = plsc.VectorSubcoreMesh(
        core_axis_name="c", subcore_axis_name="s", num_cores=NC, num_subcores=NS
    )
    return pl.kernel(
        _sc_gather_sum,
        out_type=jax.ShapeDtypeStruct((PROJ_DIM, BATCH), jnp.float32),
        mesh=mesh,
        scratch_types=[
            pltpu.VMEM((B,), jnp.float32),
            pltpu.VMEM((HB,), jnp.int32),
            pltpu.VMEM((BATCH,), jnp.float32),
            pltpu.SemaphoreType.DMA,
        ],
        compiler_params=pltpu.CompilerParams(
            use_tc_tiling_on_sc=True, needs_layout_passes=False
        ),
    )(tables_t, buckets_t)


TB = 2048


def _mlp_body(zt_ref, w1_ref, b1_ref, w2_ref, b2_ref, o_ref):
    h = lax.dot_general(
        zt_ref[...], w1_ref[...], (((0,), (0,)), ((), ())),
        preferred_element_type=jnp.float32,
    )
    h = jnp.maximum(h + b1_ref[...], 0.0)
    o = jnp.dot(h, w2_ref[...], preferred_element_type=jnp.float32)
    o_ref[...] = o + b2_ref[...]


@jax.jit
def _mlp(zt, W1, b1, W2, b2):
    return pl.pallas_call(
        _mlp_body,
        grid=(BATCH // TB,),
        in_specs=[
            pl.BlockSpec((PROJ_DIM, TB), lambda i: (0, i)),
            pl.BlockSpec((PROJ_DIM, HIDDEN), lambda i: (0, 0)),
            pl.BlockSpec((1, HIDDEN), lambda i: (0, 0)),
            pl.BlockSpec((HIDDEN, EMB_DIM), lambda i: (0, 0)),
            pl.BlockSpec((1, EMB_DIM), lambda i: (0, 0)),
        ],
        out_specs=pl.BlockSpec((TB, EMB_DIM), lambda i: (i, 0)),
        out_shape=jax.ShapeDtypeStruct((BATCH, EMB_DIM), jnp.float32),
    )(zt, W1, b1.reshape(1, HIDDEN), W2, b2.reshape(1, EMB_DIM))


def kernel(buckets, tables, W1, b1, W2, b2):
    tables_t = tables.transpose(0, 2, 1)
    buckets_t = buckets.T
    zt = _gather_sum(tables_t, buckets_t)
    return _mlp(zt, W1, b1, W2, b2)


# speedup vs baseline: 2.9368x; 1.1095x over previous
import jax
import jax.numpy as jnp
from jax import lax
from jax.experimental import pallas as pl
from jax.experimental.pallas import tpu as pltpu
from jax.experimental.pallas import tpu_sc as plsc

K = 8
B = 100000
PROJ_DIM = 32
EMB_DIM = 32
HIDDEN = 128
BATCH = 16384

NC = 2
NS = 16
NW = NC * NS
L = 16
HB = 8192
UNROLL = 8


def _sc_gather_sum(tab_hbm, bkt_hbm, zt_hbm, v_ref, bidx_v, z_v, sem):
    c = lax.axis_index("c")
    s = lax.axis_index("s")
    j = s * NC + c

    for k in range(K):
        pltpu.sync_copy(tab_hbm.at[k, j], v_ref)
        for h in range(BATCH // HB):
            pltpu.sync_copy(bkt_hbm.at[k, pl.ds(h * HB, HB)], bidx_v)

            def gather_body(g0, _, k=k, h=h):
                for u in range(UNROLL):
                    g = g0 * UNROLL + u
                    idx = bidx_v[pl.ds(g * L, L)]
                    vals = plsc.load_gather(v_ref, [idx])
                    zsl = pl.ds(h * HB + g * L, L)
                    if k == 0:
                        z_v[zsl] = vals
                    else:
                        z_v[zsl] = z_v[zsl] + vals
                return 0

            lax.fori_loop(0, HB // L // UNROLL, gather_body, 0)

    pltpu.sync_copy(z_v, zt_hbm.at[j])


@jax.jit
def _gather_sum(tables_t, buckets_t):
    mesh = plsc.VectorSubcoreMesh(
        core_axis_name="c", subcore_axis_name="s", num_cores=NC, num_subcores=NS
    )
    return pl.kernel(
        _sc_gather_sum,
        out_type=jax.ShapeDtypeStruct((PROJ_DIM, BATCH), jnp.float32),
        mesh=mesh,
        scratch_types=[
            pltpu.VMEM((B,), jnp.float32),
            pltpu.VMEM((HB,), jnp.int32),
            pltpu.VMEM((BATCH,), jnp.float32),
            pltpu.SemaphoreType.DMA,
        ],
        compiler_params=pltpu.CompilerParams(
            use_tc_tiling_on_sc=True, needs_layout_passes=False
        ),
    )(tables_t, buckets_t)


TB = 2048


def _mlp_body(zt_ref, w1_ref, b1_ref, w2_ref, b2_ref, o_ref):
    h = lax.dot_general(
        zt_ref[...], w1_ref[...], (((0,), (0,)), ((), ())),
        preferred_element_type=jnp.float32,
    )
    h = jnp.maximum(h + b1_ref[...], 0.0)
    o = jnp.dot(h, w2_ref[...], preferred_element_type=jnp.float32)
    o_ref[...] = o + b2_ref[...]


@jax.jit
def _mlp(zt, W1, b1, W2, b2):
    return pl.pallas_call(
        _mlp_body,
        grid=(BATCH // TB,),
        in_specs=[
            pl.BlockSpec((PROJ_DIM, TB), lambda i: (0, i)),
            pl.BlockSpec((PROJ_DIM, HIDDEN), lambda i: (0, 0)),
            pl.BlockSpec((1, HIDDEN), lambda i: (0, 0)),
            pl.BlockSpec((HIDDEN, EMB_DIM), lambda i: (0, 0)),
            pl.BlockSpec((1, EMB_DIM), lambda i: (0, 0)),
        ],
        out_specs=pl.BlockSpec((TB, EMB_DIM), lambda i: (i, 0)),
        out_shape=jax.ShapeDtypeStruct((BATCH, EMB_DIM), jnp.float32),
    )(zt, W1, b1.reshape(1, HIDDEN), W2, b2.reshape(1, EMB_DIM))


def kernel(buckets, tables, W1, b1, W2, b2):
    tables_t = tables.transpose(0, 2, 1)
    buckets_t = buckets.T
    zt = _gather_sum(tables_t, buckets_t)
    return _mlp(zt, W1, b1, W2, b2)
